# Initial kernel scaffold; baseline (speedup 1.0000x reference)
import jax
import jax.numpy as jnp
from jax.experimental import pallas as pl


def kernel(token_ids, weight):
    raise NotImplementedError("write your pallas kernel here")


# speedup vs baseline: 1.7979x; 1.7979x over previous
import functools

import jax
import jax.numpy as jnp
from jax import lax
from jax.experimental import pallas as pl
from jax.experimental.pallas import tpu as pltpu
from jax.experimental.pallas import tpu_sc as plsc

_NUM_CORES = 2
_NUM_SUBCORES = 16
_NW = _NUM_CORES * _NUM_SUBCORES

_GRP = 128
_CHUNK_GRPS = 4
_CHUNK = _GRP * _CHUNK_GRPS


@functools.cache
def _build(B, D):
    b_per_w = B // _NW
    n_chunks = b_per_w // _CHUNK
    mesh = plsc.VectorSubcoreMesh(
        core_axis_name="c", subcore_axis_name="s",
        num_cores=_NUM_CORES, num_subcores=_NUM_SUBCORES)

    @functools.partial(
        pl.kernel,
        out_type=jax.ShapeDtypeStruct((B, D), jnp.float32),
        mesh=mesh,
        scratch_types=[
            pltpu.VMEM((_CHUNK_GRPS, _GRP), jnp.int32),
            pltpu.VMEM((_CHUNK, D), jnp.float32),
            pltpu.SemaphoreType.DMA,
        ],
        compiler_params=pltpu.CompilerParams(use_tc_tiling_on_sc=False),
    )
    def gather_kernel(idx_hbm, table_hbm, out_hbm, idx_v, rows_v, sem):
        wid = lax.axis_index("s") * _NUM_CORES + lax.axis_index("c")
        base_grp = wid * (b_per_w // _GRP)
        base_row = wid * b_per_w

        def body(i, carry):
            pltpu.sync_copy(
                idx_hbm.at[pl.ds(base_grp + i * _CHUNK_GRPS, _CHUNK_GRPS)],
                idx_v)
            copies = [
                pltpu.async_copy(
                    table_hbm.at[idx_v.at[j]],
                    rows_v.at[pl.ds(j * _GRP, _GRP)],
                    sem)
                for j in range(_CHUNK_GRPS)]
            for c in copies:
                c.wait()
            pltpu.sync_copy(
                rows_v,
                out_hbm.at[pl.ds(base_row + i * _CHUNK, _CHUNK)])
            return carry

        lax.fori_loop(0, n_chunks, body, 0)

    return gather_kernel


def kernel(token_ids, weight):
    B = token_ids.shape[0] * token_ids.shape[1]
    D = weight.shape[1]
    idx = token_ids.reshape(B // _GRP, _GRP).astype(jnp.int32)
    out = _build(B, D)(idx, weight)
    return out.reshape(*token_ids.shape, D)


# speedup vs baseline: 1.8732x; 1.0419x over previous
import functools

import jax
import jax.numpy as jnp
from jax import lax
from jax.experimental import pallas as pl
from jax.experimental.pallas import tpu as pltpu
from jax.experimental.pallas import tpu_sc as plsc

_NUM_CORES = 2
_NUM_SUBCORES = 16
_NW = _NUM_CORES * _NUM_SUBCORES

_GRP = 128
_CHUNK_GRPS = 4
_CHUNK = _GRP * _CHUNK_GRPS


@functools.cache
def _build(B, D):
    b_per_w = B // _NW
    n_grps = b_per_w // _GRP
    n_chunks = b_per_w // _CHUNK
    assert n_chunks >= 2
    mesh = plsc.VectorSubcoreMesh(
        core_axis_name="c", subcore_axis_name="s",
        num_cores=_NUM_CORES, num_subcores=_NUM_SUBCORES)

    @functools.partial(
        pl.kernel,
        out_type=jax.ShapeDtypeStruct((B, D), jnp.float32),
        mesh=mesh,
        scratch_types=[
            pltpu.VMEM((n_grps, _GRP), jnp.int32),
            pltpu.VMEM((2, _CHUNK, D), jnp.float32),
            pltpu.SemaphoreType.DMA((2,)),
            pltpu.SemaphoreType.DMA((2,)),
        ],
        compiler_params=pltpu.CompilerParams(use_tc_tiling_on_sc=False),
    )
    def gather_kernel(idx_hbm, table_hbm, out_hbm, idx_v, rows_v, gsem, ssem):
        wid = lax.axis_index("s") * _NUM_CORES + lax.axis_index("c")
        base_grp = wid * n_grps
        base_row = wid * b_per_w

        pltpu.sync_copy(idx_hbm.at[pl.ds(base_grp, n_grps)], idx_v)

        def issue_gathers(i, p):
            for j in range(_CHUNK_GRPS):
                pltpu.async_copy(
                    table_hbm.at[idx_v.at[i * _CHUNK_GRPS + j]],
                    rows_v.at[p, pl.ds(j * _GRP, _GRP)],
                    gsem.at[p])

        def wait_gathers(p):
            pltpu.make_async_copy(
                table_hbm.at[pl.ds(0, _CHUNK)], rows_v.at[p], gsem.at[p]
            ).wait()

        def issue_store(i, p):
            pltpu.async_copy(
                rows_v.at[p],
                out_hbm.at[pl.ds(base_row + i * _CHUNK, _CHUNK)],
                ssem.at[p])

        def wait_store(p):
            pltpu.make_async_copy(
                rows_v.at[p], out_hbm.at[pl.ds(base_row, _CHUNK)], ssem.at[p]
            ).wait()

        issue_gathers(0, 0)
        issue_gathers(1, 1)
        wait_gathers(0)
        issue_store(0, 0)

        def body(i, carry):
            p = lax.rem(i, 2)
            q = 1 - p
            wait_store(p)
            issue_gathers(i, p)
            wait_gathers(q)
            issue_store(i - 1, q)
            return carry

        lax.fori_loop(2, n_chunks, body, 0)

        last = n_chunks - 1
        p_last = last % 2
        wait_gathers(p_last)
        issue_store(last, p_last)
        wait_store(0)
        wait_store(1)

    return gather_kernel


def kernel(token_ids, weight):
    B = token_ids.shape[0] * token_ids.shape[1]
    D = weight.shape[1]
    idx = token_ids.reshape(B // _GRP, _GRP).astype(jnp.int32)
    out = _build(B, D)(idx, weight)
    return out.reshape(*token_ids.shape, D)
